# Initial kernel scaffold; baseline (speedup 1.0000x reference)
import functools

import jax
import jax.numpy as jnp
from jax.experimental import pallas as pl
from jax.experimental.pallas import tpu as pltpu
from jax.experimental.pallas import tpu_sc as plsc

DIM = 64
W = 128


def _sc_gather(out_embed, idx_all, in_embed, idx_c):
    n_all = idx_all.shape[1]
    n_c = idx_c.shape[1]
    mesh = plsc.VectorSubcoreMesh(core_axis_name="c", subcore_axis_name="s")

    @functools.partial(
        pl.kernel,
        out_type=(
            jax.ShapeDtypeStruct((n_all, DIM), jnp.float32),
            jax.ShapeDtypeStruct((n_c, DIM), jnp.float32),
        ),
        mesh=mesh,
    )
    def gather_kernel(out_hbm, idx_all_hbm, in_hbm, idx_c_hbm, rows_hbm, vc_hbm):
        def body_out(i_vmem, o_vmem):
            pltpu.sync_copy(out_hbm.at[i_vmem.at[0]], o_vmem)

        pltpu.emit_pipeline(
            body_out,
            grid=(n_all // W,),
            in_specs=[pl.BlockSpec((1, W), index_map=lambda i: (0, i))],
            out_specs=[pl.BlockSpec((W, DIM), index_map=lambda i: (i, 0))],
            core_axis_name=("c", "s"),
            dimension_semantics=(pltpu.PARALLEL,),
        )(idx_all_hbm, rows_hbm)

        def body_in(i_vmem, o_vmem):
            pltpu.sync_copy(in_hbm.at[i_vmem.at[0]], o_vmem)

        pltpu.emit_pipeline(
            body_in,
            grid=(n_c // W,),
            in_specs=[pl.BlockSpec((1, W), index_map=lambda i: (0, i))],
            out_specs=[pl.BlockSpec((W, DIM), index_map=lambda i: (i, 0))],
            core_axis_name=("c", "s"),
            dimension_semantics=(pltpu.PARALLEL,),
        )(idx_c_hbm, vc_hbm)

    return gather_kernel(out_embed, idx_all, in_embed, idx_c)


def _tc_loss(rows3, vc):
    k1, b, _ = rows3.shape
    blk = 512
    n_blocks = b // blk

    def body(rows_ref, vc_ref, out_ref, acc_ref):
        i = pl.program_id(0)

        @pl.when(i == 0)
        def _():
            acc_ref[0] = 0.0

        vcb = vc_ref[...]
        pos = jnp.sum(rows_ref[0] * vcb, axis=1)
        total = jnp.sum(jnp.log(jax.nn.sigmoid(pos)))
        for k in range(1, k1):
            nd = jnp.sum(rows_ref[k] * vcb, axis=1)
            total += jnp.sum(jnp.log(jax.nn.sigmoid(-nd)))
        acc_ref[0] += total

        @pl.when(i == n_blocks - 1)
        def _():
            out_ref[0, 0] = -acc_ref[0] / b

    out = pl.pallas_call(
        body,
        grid=(n_blocks,),
        in_specs=[
            pl.BlockSpec((k1, blk, DIM), lambda i: (0, i, 0)),
            pl.BlockSpec((blk, DIM), lambda i: (i, 0)),
        ],
        out_specs=pl.BlockSpec(memory_space=pltpu.SMEM),
        out_shape=jax.ShapeDtypeStruct((1, 1), jnp.float32),
        scratch_shapes=[pltpu.SMEM((1,), jnp.float32)],
    )(rows3, vc)
    return out[0, 0]


def kernel(centers, contexts, negatives, in_embed, out_embed):
    b = centers.shape[0]
    k1 = 1 + negatives.shape[1]
    idx_all = jnp.concatenate([contexts[None, :], negatives.T], axis=0)
    idx_all = idx_all.reshape(1, -1).astype(jnp.int32)
    idx_c = centers[None, :].astype(jnp.int32)
    rows, vc = _sc_gather(out_embed, idx_all, in_embed, idx_c)
    rows3 = rows.reshape(k1, b, DIM)
    return _tc_loss(rows3, vc)


# speedup vs baseline: 4.3236x; 4.3236x over previous
import functools

import jax
import jax.numpy as jnp
from jax.experimental import pallas as pl
from jax.experimental.pallas import tpu as pltpu
from jax.experimental.pallas import tpu_sc as plsc

DIM = 64
W = 128


def _sc_gather(out_embed, idx_all, in_embed, idx_c):
    n_all = idx_all.shape[1]
    n_c = idx_c.shape[1]
    mesh = plsc.VectorSubcoreMesh(core_axis_name="c", subcore_axis_name="s")

    @functools.partial(
        pl.kernel,
        out_type=(
            jax.ShapeDtypeStruct((n_all, DIM), jnp.float32),
            jax.ShapeDtypeStruct((n_c, DIM), jnp.float32),
        ),
        mesh=mesh,
        compiler_params=pltpu.CompilerParams(use_tc_tiling_on_sc=False),
    )
    def gather_kernel(out_hbm, idx_all_hbm, in_hbm, idx_c_hbm, rows_hbm, vc_hbm):
        def body_out(i_vmem, o_vmem):
            pltpu.sync_copy(out_hbm.at[i_vmem.at[0]], o_vmem)

        pltpu.emit_pipeline(
            body_out,
            grid=(n_all // W,),
            in_specs=[pl.BlockSpec((1, W), index_map=lambda i: (0, i))],
            out_specs=[pl.BlockSpec((W, DIM), index_map=lambda i: (i, 0))],
            core_axis_name=("c", "s"),
            dimension_semantics=(pltpu.PARALLEL,),
        )(idx_all_hbm, rows_hbm)

        def body_in(i_vmem, o_vmem):
            pltpu.sync_copy(in_hbm.at[i_vmem.at[0]], o_vmem)

        pltpu.emit_pipeline(
            body_in,
            grid=(n_c // W,),
            in_specs=[pl.BlockSpec((1, W), index_map=lambda i: (0, i))],
            out_specs=[pl.BlockSpec((W, DIM), index_map=lambda i: (i, 0))],
            core_axis_name=("c", "s"),
            dimension_semantics=(pltpu.PARALLEL,),
        )(idx_c_hbm, vc_hbm)

    return gather_kernel(out_embed, idx_all, in_embed, idx_c)


def _tc_loss(rows3, vc):
    k1, b, _ = rows3.shape
    blk = 512
    n_blocks = b // blk

    def body(rows_ref, vc_ref, out_ref, acc_ref):
        i = pl.program_id(0)

        @pl.when(i == 0)
        def _():
            acc_ref[0] = 0.0

        vcb = vc_ref[...]
        pos = jnp.sum(rows_ref[0] * vcb, axis=1)
        total = jnp.sum(jnp.log(jax.nn.sigmoid(pos)))
        for k in range(1, k1):
            nd = jnp.sum(rows_ref[k] * vcb, axis=1)
            total += jnp.sum(jnp.log(jax.nn.sigmoid(-nd)))
        acc_ref[0] += total

        @pl.when(i == n_blocks - 1)
        def _():
            out_ref[0, 0] = -acc_ref[0] / b

    out = pl.pallas_call(
        body,
        grid=(n_blocks,),
        in_specs=[
            pl.BlockSpec((k1, blk, DIM), lambda i: (0, i, 0)),
            pl.BlockSpec((blk, DIM), lambda i: (i, 0)),
        ],
        out_specs=pl.BlockSpec(memory_space=pltpu.SMEM),
        out_shape=jax.ShapeDtypeStruct((1, 1), jnp.float32),
        scratch_shapes=[pltpu.SMEM((1,), jnp.float32)],
    )(rows3, vc)
    return out[0, 0]


def kernel(centers, contexts, negatives, in_embed, out_embed):
    b = centers.shape[0]
    k1 = 1 + negatives.shape[1]
    idx_all = jnp.concatenate([contexts[None, :], negatives.T], axis=0)
    idx_all = idx_all.reshape(1, -1).astype(jnp.int32)
    idx_c = centers[None, :].astype(jnp.int32)
    rows, vc = _sc_gather(out_embed, idx_all, in_embed, idx_c)
    rows3 = rows.reshape(k1, b, DIM)
    return _tc_loss(rows3, vc)


# speedup vs baseline: 7.2080x; 1.6671x over previous
import functools

import jax
import jax.numpy as jnp
from jax.experimental import pallas as pl
from jax.experimental.pallas import tpu as pltpu
from jax.experimental.pallas import tpu_sc as plsc

DIM = 64
W = 128
WV = 4096
HV = WV // 2


def _tc_transpose(table):
    v = table.shape[0]
    n_blk = pl.cdiv(v, WV)
    t_t = jnp.swapaxes(table, 0, 1)

    def body(in_ref, out_ref):
        tr = jnp.transpose(in_ref[...])
        out_ref[:, :DIM] = tr[:HV]
        out_ref[:, DIM:] = tr[HV:]

    return pl.pallas_call(
        body,
        grid=(n_blk,),
        in_specs=[pl.BlockSpec((DIM, WV), lambda i: (0, i))],
        out_specs=pl.BlockSpec((HV, 128), lambda i: (i, 0)),
        out_shape=jax.ShapeDtypeStruct((n_blk * HV, 128), jnp.float32),
    )(t_t)


def _sigma(idx):
    i = idx // WV
    r = idx % WV
    return i * WV + jnp.where(r < HV, 2 * r, 2 * (r - HV) + 1)


def _sc_gather(table_lin, idx, n_rows):
    mesh = plsc.VectorSubcoreMesh(core_axis_name="c", subcore_axis_name="s")

    @functools.partial(
        pl.kernel,
        out_type=jax.ShapeDtypeStruct((n_rows, DIM), jnp.float32),
        mesh=mesh,
        compiler_params=pltpu.CompilerParams(use_tc_tiling_on_sc=False),
    )
    def gather_kernel(table_hbm, idx_hbm, rows_hbm):
        def body(i_vmem, o_vmem):
            pltpu.sync_copy(table_hbm.at[i_vmem.at[0]], o_vmem)

        pltpu.emit_pipeline(
            body,
            grid=(n_rows // W,),
            in_specs=[pl.BlockSpec((1, W), index_map=lambda i: (0, i))],
            out_specs=[pl.BlockSpec((W, DIM), index_map=lambda i: (i, 0))],
            core_axis_name=("c", "s"),
            dimension_semantics=(pltpu.PARALLEL,),
        )(idx_hbm, rows_hbm)

    return gather_kernel(table_lin, idx)


def _tc_loss(rows3, vc2):
    k1, half_b, _ = rows3.shape
    b = half_b * 2
    blk = 256
    n_blocks = half_b // blk

    def body(rows_ref, vc_ref, out_ref, acc_ref):
        i = pl.program_id(0)

        @pl.when(i == 0)
        def _():
            acc_ref[0] = 0.0

        vcb = vc_ref[...]
        total = 0.0
        for k in range(k1):
            prod = rows_ref[k] * vcb
            d_a = jnp.sum(prod[:, :DIM], axis=1)
            d_b = jnp.sum(prod[:, DIM:], axis=1)
            if k == 0:
                total += jnp.sum(jnp.log(jax.nn.sigmoid(d_a)))
                total += jnp.sum(jnp.log(jax.nn.sigmoid(d_b)))
            else:
                total += jnp.sum(jnp.log(jax.nn.sigmoid(-d_a)))
                total += jnp.sum(jnp.log(jax.nn.sigmoid(-d_b)))
        acc_ref[0] += total

        @pl.when(i == n_blocks - 1)
        def _():
            out_ref[0, 0] = -acc_ref[0] / b

    out = pl.pallas_call(
        body,
        grid=(n_blocks,),
        in_specs=[
            pl.BlockSpec((k1, blk, 128), lambda i: (0, i, 0)),
            pl.BlockSpec((blk, 128), lambda i: (i, 0)),
        ],
        out_specs=pl.BlockSpec(memory_space=pltpu.SMEM),
        out_shape=jax.ShapeDtypeStruct((1, 1), jnp.float32),
        scratch_shapes=[pltpu.SMEM((1,), jnp.float32)],
    )(rows3, vc2)
    return out[0, 0]


def kernel(centers, contexts, negatives, in_embed, out_embed):
    b = centers.shape[0]
    k1 = 1 + negatives.shape[1]
    v = in_embed.shape[0]
    n_all = k1 * b

    idx_all = jnp.concatenate([contexts[None, :], negatives.T], axis=0)
    idx_all = _sigma(idx_all.reshape(1, -1).astype(jnp.int32))
    idx_c = _sigma(centers[None, :].astype(jnp.int32))

    out_packed = _tc_transpose(out_embed)
    out_lin = out_packed.reshape(out_packed.shape[0] * 2, DIM)
    rows = _sc_gather(out_lin, idx_all, n_all)
    in_packed = _tc_transpose(in_embed)
    in_lin = in_packed.reshape(in_packed.shape[0] * 2, DIM)
    vc = _sc_gather(in_lin, idx_c, b)

    rows3 = rows.reshape(k1, b // 2, 128)
    vc2 = vc.reshape(b // 2, 128)
    return _tc_loss(rows3, vc2)


# speedup vs baseline: 8.4068x; 1.1663x over previous
import functools

import jax
import jax.numpy as jnp
from jax.experimental import pallas as pl
from jax.experimental.pallas import tpu as pltpu
from jax.experimental.pallas import tpu_sc as plsc

DIM = 64
W = 128
WV = 4096
HV = WV // 2


def _tc_transpose(table):
    v = table.shape[0]
    n_blk = pl.cdiv(v, WV)
    t_t = jnp.swapaxes(table, 0, 1)

    def body(in_ref, out_ref):
        tr = jnp.transpose(in_ref[...])
        out_ref[:, :DIM] = tr[:HV]
        out_ref[:, DIM:] = tr[HV:]

    return pl.pallas_call(
        body,
        grid=(n_blk,),
        in_specs=[pl.BlockSpec((DIM, WV), lambda i: (0, i))],
        out_specs=pl.BlockSpec((HV, 128), lambda i: (i, 0)),
        out_shape=jax.ShapeDtypeStruct((n_blk * HV, 128), jnp.float32),
        compiler_params=pltpu.CompilerParams(
            dimension_semantics=("parallel",)),
    )(t_t)


def _sigma(idx):
    i = idx // WV
    r = idx % WV
    return i * WV + jnp.where(r < HV, 2 * r, 2 * (r - HV) + 1)


def _sc_gather(table_lin, idx, n_rows):
    mesh = plsc.VectorSubcoreMesh(core_axis_name="c", subcore_axis_name="s")

    @functools.partial(
        pl.kernel,
        out_type=jax.ShapeDtypeStruct((n_rows, DIM), jnp.float32),
        mesh=mesh,
        compiler_params=pltpu.CompilerParams(use_tc_tiling_on_sc=False),
    )
    def gather_kernel(table_hbm, idx_hbm, rows_hbm):
        def body(i_vmem, o_vmem):
            pltpu.sync_copy(table_hbm.at[i_vmem.at[0]], o_vmem)

        pltpu.emit_pipeline(
            body,
            grid=(n_rows // W,),
            in_specs=[pl.BlockSpec((1, W), index_map=lambda i: (0, i))],
            out_specs=[pl.BlockSpec((W, DIM), index_map=lambda i: (i, 0))],
            core_axis_name=("c", "s"),
            dimension_semantics=(pltpu.PARALLEL,),
        )(idx_hbm, rows_hbm)

    return gather_kernel(table_lin, idx)


def _tc_loss(rows3, vc2):
    k1, half_b, _ = rows3.shape
    blk = 256
    n_blocks = half_b // blk

    def body(rows_ref, vc_ref, out_ref):
        vc_t = jnp.transpose(vc_ref[...])
        ds = []
        for k in range(k1):
            prod_t = jnp.transpose(rows_ref[k]) * vc_t
            d_a = jnp.sum(prod_t[:DIM], axis=0)
            d_b = jnp.sum(prod_t[DIM:], axis=0)
            sgn = 1.0 if k == 0 else -1.0
            ds.append(sgn * d_a)
            ds.append(sgn * d_b)
        dmat = jnp.stack(ds)
        out_ref[0, 0, 0] = jnp.sum(jnp.log(jax.nn.sigmoid(dmat)))

    out = pl.pallas_call(
        body,
        grid=(n_blocks,),
        in_specs=[
            pl.BlockSpec((k1, blk, 128), lambda i: (0, i, 0)),
            pl.BlockSpec((blk, 128), lambda i: (i, 0)),
        ],
        out_specs=pl.BlockSpec(
            (1, 1, 1), lambda i: (i, 0, 0), memory_space=pltpu.SMEM),
        out_shape=jax.ShapeDtypeStruct((n_blocks, 1, 1), jnp.float32),
        compiler_params=pltpu.CompilerParams(
            dimension_semantics=("parallel",)),
    )(rows3, vc2)
    return out


def kernel(centers, contexts, negatives, in_embed, out_embed):
    b = centers.shape[0]
    k1 = 1 + negatives.shape[1]
    v = in_embed.shape[0]
    n_all = k1 * b

    idx_all = jnp.concatenate([contexts[None, :], negatives.T], axis=0)
    idx_all = _sigma(idx_all.reshape(1, -1).astype(jnp.int32))
    idx_c = _sigma(centers[None, :].astype(jnp.int32))

    out_packed = _tc_transpose(out_embed)
    out_lin = out_packed.reshape(out_packed.shape[0] * 2, DIM)
    rows = _sc_gather(out_lin, idx_all, n_all)
    in_packed = _tc_transpose(in_embed)
    in_lin = in_packed.reshape(in_packed.shape[0] * 2, DIM)
    vc = _sc_gather(in_lin, idx_c, b)

    rows3 = rows.reshape(k1, b // 2, 128)
    vc2 = vc.reshape(b // 2, 128)
    partials = _tc_loss(rows3, vc2)
    return -jnp.sum(partials) / b
